# baseline (speedup 1.0000x reference)
import functools
import math

import jax
import jax.numpy as jnp
import numpy as np
from jax import lax
from jax.experimental import pallas as pl
from jax.experimental.pallas import tpu as pltpu
from jax.experimental.pallas import tpu_sc as plsc

NTOKENS = 100000
NINP = 128
SEQ = 2048
BATCH = 16
SCALE = math.sqrt(float(NINP))

NW = 32
S_PER_W = SEQ // NW
LANE_GROUPS = NINP // 16

S_PER_CH = 4
CH = S_PER_CH * BATCH
NCH = S_PER_W // S_PER_CH
NBUF = 4
NITER = NCH // NBUF


def _make_pe() -> np.ndarray:
    position = np.arange(SEQ, dtype=np.float32)[:, None]
    div_term = np.exp(
        np.arange(0, NINP, 2, dtype=np.float32) * (-math.log(10000.0) / NINP)
    )
    pe = np.zeros((SEQ, NINP), dtype=np.float32)
    pe[:, 0::2] = np.sin(position * div_term)
    pe[:, 1::2] = np.cos(position * div_term)
    return pe


_PE = _make_pe()

_mesh = plsc.VectorSubcoreMesh(core_axis_name="c", subcore_axis_name="s")


def _compute_chunk(c, gbuf, obuf, pe_v):
    for sj in range(S_PER_CH):
        prow = c * S_PER_CH + sj
        pvals = [pe_v[prow, pl.ds(16 * j, 16)] for j in range(LANE_GROUPS)]

        def bb_body(bb, carry, sj=sj, pvals=pvals):
            r = sj * BATCH + bb
            for j in range(LANE_GROUPS):
                obuf[r, pl.ds(16 * j, 16)] = (
                    gbuf[r, pl.ds(16 * j, 16)] * SCALE + pvals[j]
                )
            return carry

        lax.fori_loop(0, BATCH, bb_body, 0)


@functools.partial(
    pl.kernel,
    mesh=_mesh,
    out_type=jax.ShapeDtypeStruct((SEQ * BATCH, NINP), jnp.float32),
    scratch_types=(
        [pltpu.VMEM((S_PER_W, BATCH), jnp.int32)]
        + [pltpu.VMEM((S_PER_W * BATCH,), jnp.int32)]
        + [pltpu.VMEM((S_PER_W, NINP), jnp.float32)]
        + [pltpu.VMEM((CH, NINP), jnp.float32)] * NBUF
        + [pltpu.VMEM((CH, NINP), jnp.float32)] * NBUF
        + [pltpu.SemaphoreType.DMA] * (2 * NBUF)
    ),
)
def _sc_embed(x_hbm, pe_hbm, table_hbm, out_hbm, idx_s, idx_v, pe_v, *rest):
    gbufs = rest[0:NBUF]
    obufs = rest[NBUF : 2 * NBUF]
    gsems = rest[2 * NBUF : 3 * NBUF]
    osems = rest[3 * NBUF : 4 * NBUF]

    wid = lax.axis_index("s") * 2 + lax.axis_index("c")
    sbase = wid * S_PER_W

    pltpu.sync_copy(x_hbm.at[pl.ds(sbase, S_PER_W)], idx_s)
    pltpu.sync_copy(pe_hbm.at[pl.ds(sbase, S_PER_W)], pe_v)
    for r in range(S_PER_W):
        idx_v[pl.ds(r * BATCH, BATCH)] = idx_s[r, pl.ds(0, BATCH)]

    def _gather(c, b):
        return pltpu.async_copy(
            table_hbm.at[idx_v.at[pl.ds(c * CH, CH)]],
            gbufs[b],
            gsems[b],
        )

    def _writeback(c, b):
        return pltpu.async_copy(
            obufs[b],
            out_hbm.at[pl.ds(sbase * BATCH + c * CH, CH)],
            osems[b],
        )

    for b in range(NBUF):
        _gather(b, b)

    def g_body(g, carry):
        for b in range(NBUF):
            c = g * NBUF + b
            pltpu.make_async_copy(
                table_hbm.at[idx_v.at[pl.ds(0, CH)]], gbufs[b], gsems[b]
            ).wait()

            @pl.when(g > 0)
            def _drain_prev(b=b):
                pltpu.make_async_copy(
                    obufs[b], out_hbm.at[pl.ds(0, CH)], osems[b]
                ).wait()

            _compute_chunk(c, gbufs[b], obufs[b], pe_v)

            @pl.when(g < NITER - 1)
            def _issue_next(b=b, c=c):
                _gather(c + NBUF, b)

            _writeback(c, b)
        return carry

    lax.fori_loop(0, NITER, g_body, 0)

    for b in range(NBUF):
        pltpu.make_async_copy(
            obufs[b], out_hbm.at[pl.ds(0, CH)], osems[b]
        ).wait()


def kernel(x, table):
    pe = jnp.asarray(_PE)
    out = _sc_embed(x, pe, table)
    return out.reshape(SEQ, BATCH, NINP)


# speedup vs baseline: 1.0220x; 1.0220x over previous
import functools
import math

import jax
import jax.numpy as jnp
import numpy as np
from jax import lax
from jax.experimental import pallas as pl
from jax.experimental.pallas import tpu as pltpu
from jax.experimental.pallas import tpu_sc as plsc

NTOKENS = 100000
NINP = 128
SEQ = 2048
BATCH = 16
SCALE = math.sqrt(float(NINP))

NW = 32
S_PER_W = SEQ // NW
LANE_GROUPS = NINP // 16

S_PER_CH = 2
CH = S_PER_CH * BATCH
NCH = S_PER_W // S_PER_CH
NBUF = 8
NITER = NCH // NBUF


def _make_pe() -> np.ndarray:
    position = np.arange(SEQ, dtype=np.float32)[:, None]
    div_term = np.exp(
        np.arange(0, NINP, 2, dtype=np.float32) * (-math.log(10000.0) / NINP)
    )
    pe = np.zeros((SEQ, NINP), dtype=np.float32)
    pe[:, 0::2] = np.sin(position * div_term)
    pe[:, 1::2] = np.cos(position * div_term)
    return pe


_PE = _make_pe()

_mesh = plsc.VectorSubcoreMesh(core_axis_name="c", subcore_axis_name="s")


def _compute_chunk(c, gbuf, obuf, pe_v):
    for sj in range(S_PER_CH):
        prow = c * S_PER_CH + sj
        pvals = [pe_v[prow, pl.ds(16 * j, 16)] for j in range(LANE_GROUPS)]

        def bb_body(bb, carry, sj=sj, pvals=pvals):
            r = sj * BATCH + bb
            for j in range(LANE_GROUPS):
                obuf[r, pl.ds(16 * j, 16)] = (
                    gbuf[r, pl.ds(16 * j, 16)] * SCALE + pvals[j]
                )
            return carry

        lax.fori_loop(0, BATCH, bb_body, 0)


@functools.partial(
    pl.kernel,
    mesh=_mesh,
    out_type=jax.ShapeDtypeStruct((SEQ * BATCH, NINP), jnp.float32),
    scratch_types=(
        [pltpu.VMEM((S_PER_W, BATCH), jnp.int32)]
        + [pltpu.VMEM((S_PER_W * BATCH,), jnp.int32)]
        + [pltpu.VMEM((S_PER_W, NINP), jnp.float32)]
        + [pltpu.VMEM((CH, NINP), jnp.float32)] * NBUF
        + [pltpu.VMEM((CH, NINP), jnp.float32)] * NBUF
        + [pltpu.SemaphoreType.DMA] * (2 * NBUF)
    ),
)
def _sc_embed(x_hbm, pe_hbm, table_hbm, out_hbm, idx_s, idx_v, pe_v, *rest):
    gbufs = rest[0:NBUF]
    obufs = rest[NBUF : 2 * NBUF]
    gsems = rest[2 * NBUF : 3 * NBUF]
    osems = rest[3 * NBUF : 4 * NBUF]

    wid = lax.axis_index("s") * 2 + lax.axis_index("c")
    sbase = wid * S_PER_W

    pltpu.sync_copy(x_hbm.at[pl.ds(sbase, S_PER_W)], idx_s)
    for r in range(S_PER_W):
        idx_v[pl.ds(r * BATCH, BATCH)] = idx_s[r, pl.ds(0, BATCH)]

    def _gather(c, b):
        return pltpu.async_copy(
            table_hbm.at[idx_v.at[pl.ds(c * CH, CH)]],
            gbufs[b],
            gsems[b],
        )

    def _writeback(c, b):
        return pltpu.async_copy(
            obufs[b],
            out_hbm.at[pl.ds(sbase * BATCH + c * CH, CH)],
            osems[b],
        )

    for b in range(NBUF):
        _gather(b, b)
    pltpu.sync_copy(pe_hbm.at[pl.ds(sbase, S_PER_W)], pe_v)

    def g_body(g, carry):
        for b in range(NBUF):
            c = g * NBUF + b
            pltpu.make_async_copy(
                table_hbm.at[idx_v.at[pl.ds(0, CH)]], gbufs[b], gsems[b]
            ).wait()

            @pl.when(g > 0)
            def _drain_prev(b=b):
                pltpu.make_async_copy(
                    obufs[b], out_hbm.at[pl.ds(0, CH)], osems[b]
                ).wait()

            _compute_chunk(c, gbufs[b], obufs[b], pe_v)

            @pl.when(g < NITER - 1)
            def _issue_next(b=b, c=c):
                _gather(c + NBUF, b)

            _writeback(c, b)
        return carry

    lax.fori_loop(0, NITER, g_body, 0)

    for b in range(NBUF):
        pltpu.make_async_copy(
            obufs[b], out_hbm.at[pl.ds(0, CH)], osems[b]
        ).wait()


def kernel(x, table):
    pe = jnp.asarray(_PE)
    out = _sc_embed(x, pe, table)
    return out.reshape(SEQ, BATCH, NINP)
